# baseline (speedup 1.0000x reference)
import functools

import jax
import jax.numpy as jnp
from jax import lax
from jax.experimental import pallas as pl
from jax.experimental.pallas import tpu as pltpu
from jax.experimental.pallas import tpu_sc as plsc

VOCAB = 1000
BATCH, TIME = 1024, 20
NUM_CORES = 2
NUM_SUBCORES = 16
NW = NUM_CORES * NUM_SUBCORES
NBT = 8
NVT = 125
NCHUNK = 8
T_PER_W = TIME * NBT // NW
NQ = T_PER_W * NCHUNK

_MESH = plsc.VectorSubcoreMesh(core_axis_name="c", subcore_axis_name="s")


@functools.partial(
    pl.kernel,
    mesh=_MESH,
    out_type=jax.ShapeDtypeStruct((TIME, NVT, NBT, 8, 128), jnp.float32),
    scratch_types=[
        pltpu.VMEM((TIME * 128,), jnp.int32),
        pltpu.VMEM((128, 128), jnp.float32),
        pltpu.VMEM((128, 128), jnp.float32),
        pltpu.VMEM((128, 129), jnp.float32),
        pltpu.VMEM((16, 8, 128), jnp.float32),
        pltpu.VMEM((16, 8, 128), jnp.float32),
        pltpu.SemaphoreType.DMA,
        pltpu.SemaphoreType.DMA,
        pltpu.SemaphoreType.DMA,
        pltpu.SemaphoreType.DMA,
    ],
    compiler_params=pltpu.CompilerParams(
        use_tc_tiling_on_sc=False, needs_layout_passes=False
    ),
)
def _gather_t(
    idx_hbm, tv_hbm, w_hbm, idx_v, gbuf0, gbuf1, sbuf, obuf0, obuf1,
    g0, g1, w0, w1
):
    wid = lax.axis_index("s") * NUM_CORES + lax.axis_index("c")
    bb = wid % NBT
    tg = wid // NBT
    pltpu.sync_copy(idx_hbm.at[bb], idx_v)

    lanes = lax.iota(jnp.int32, 16)
    rvecs = [m * 16 + lanes for m in range(8)]

    gbufs = (gbuf0, gbuf1)
    gsems = (g0, g1)

    def _issue(q, buf, sem):
        t = tg * T_PER_W + q // NCHUNK
        k = q % NCHUNK
        return pltpu.async_copy(
            tv_hbm.at[k].at[idx_v.at[pl.ds(t * 128, 128)]], buf, sem
        )

    _issue(0, gbufs[0], gsems[0])

    obufs = (obuf0, obuf1)
    wsems = (w0, w1)

    def _wcopy16(q, ob, wsem):
        t = tg * T_PER_W + q // NCHUNK
        k = q % NCHUNK
        return pltpu.make_async_copy(
            ob.at[pl.ds(0, 16)], w_hbm.at[t, pl.ds(k * 16, 16), bb], wsem
        )

    def _wcopy13(q, ob, wsem):
        t = tg * T_PER_W + q // NCHUNK
        return pltpu.make_async_copy(
            ob.at[pl.ds(0, 13)], w_hbm.at[t, pl.ds(112, 13), bb], wsem
        )

    def body(qo):
        for kk in range(2):
            q = qo + kk
            buf = gbufs[kk]
            sem = gsems[kk]
            ob = obufs[kk]
            wsem = wsems[kk]
            t = tg * T_PER_W + q // NCHUNK
            k = q % NCHUNK
            pltpu.make_async_copy(
                tv_hbm.at[k].at[idx_v.at[pl.ds(t * 128, 128)]], buf, sem
            ).wait()
            nq = q + 1

            @pl.when(nq < NQ)
            def _():
                _issue(nq, gbufs[1 - kk], gsems[1 - kk])

            @pl.when(q >= 2)
            def _():
                k2 = (q - 2) % NCHUNK

                @pl.when(k2 < NCHUNK - 1)
                def _():
                    _wcopy16(q - 2, ob, wsem).wait()

                @pl.when(k2 == NCHUNK - 1)
                def _():
                    _wcopy13(q - 2, ob, wsem).wait()

            def rbody(r0):
                for rr in range(4):
                    r = r0 + rr
                    vals = [buf[r, pl.ds(m * 16, 16)] for m in range(8)]
                    for m in range(8):
                        sbuf.at[r][pl.ds(m * 16, 16)] = vals[m]

            pl.loop(0, 128, step=4)(rbody)

            def tbody(vtj):
                for vs in range(8):
                    c = jnp.broadcast_to(vtj * 8 + vs, (16,)).astype(jnp.int32)
                    vals = [
                        plsc.load_gather(sbuf, [rvecs[m], c]) for m in range(8)
                    ]
                    for m in range(8):
                        ob.at[vtj, vs][pl.ds(m * 16, 16)] = vals[m]

            pl.loop(0, 16)(tbody)

            @pl.when(k < NCHUNK - 1)
            def _():
                _wcopy16(q, ob, wsem).start()

            @pl.when(k == NCHUNK - 1)
            def _():
                _wcopy13(q, ob, wsem).start()

    pl.loop(0, NQ, step=2)(body)

    for q in (NQ - 2, NQ - 1):
        kk = q % 2
        if q % NCHUNK == NCHUNK - 1:
            _wcopy13(q, obufs[kk], wsems[kk]).wait()
        else:
            _wcopy16(q, obufs[kk], wsems[kk]).wait()


def kernel(idx, table):
    idx_f = (
        idx.astype(jnp.int32)
        .T.reshape(TIME, NBT, 128)
        .transpose(1, 0, 2)
        .reshape(NBT, TIME * 128)
    )
    tv = (
        jnp.pad(table, ((0, 0), (0, 24)))
        .reshape(VOCAB, NCHUNK, 128)
        .transpose(1, 0, 2)
    )
    w = _gather_t(idx_f, tv)
    return w.transpose(2, 4, 0, 1, 3).reshape(BATCH, TIME, VOCAB)


# speedup vs baseline: 1.3884x; 1.3884x over previous
import functools

import jax
import jax.numpy as jnp
from jax import lax
from jax.experimental import pallas as pl
from jax.experimental.pallas import tpu as pltpu
from jax.experimental.pallas import tpu_sc as plsc

VOCAB = 1000
BATCH, TIME = 1024, 20
NUM_CORES = 2
NUM_SUBCORES = 16
NW = NUM_CORES * NUM_SUBCORES
NBT = 8
NVT = 125
NCHUNK = 8
T_PER_W = TIME * NBT // NW
NQ = T_PER_W * NCHUNK

_MESH = plsc.VectorSubcoreMesh(core_axis_name="c", subcore_axis_name="s")


@functools.partial(
    pl.kernel,
    mesh=_MESH,
    out_type=jax.ShapeDtypeStruct((TIME, NVT, NBT, 8, 128), jnp.float32),
    scratch_types=[
        pltpu.VMEM((TIME * 128,), jnp.int32),
        pltpu.VMEM((128, 128), jnp.float32),
        pltpu.VMEM((128, 128), jnp.float32),
        pltpu.VMEM((128, 129), jnp.float32),
        pltpu.VMEM((16, 8, 128), jnp.float32),
        pltpu.VMEM((16, 8, 128), jnp.float32),
        pltpu.SemaphoreType.DMA,
        pltpu.SemaphoreType.DMA,
        pltpu.SemaphoreType.DMA,
        pltpu.SemaphoreType.DMA,
    ],
    compiler_params=pltpu.CompilerParams(
        use_tc_tiling_on_sc=False, needs_layout_passes=False
    ),
)
def _gather_t(
    idx_hbm, tv_hbm, w_hbm, idx_v, gbuf0, gbuf1, sbuf, obuf0, obuf1,
    g0, g1, w0, w1
):
    wid = lax.axis_index("s") * NUM_CORES + lax.axis_index("c")
    bb = wid % NBT
    tg = wid // NBT
    pltpu.sync_copy(idx_hbm.at[bb], idx_v)

    lanes = lax.iota(jnp.int32, 16)
    perms = [(lanes + d) % 16 for d in range(16)]
    vt0 = lanes // 8
    vs0 = lanes % 8

    gbufs = (gbuf0, gbuf1)
    gsems = (g0, g1)

    def _issue(q, buf, sem):
        t = tg * T_PER_W + q // NCHUNK
        k = q % NCHUNK
        return pltpu.async_copy(
            tv_hbm.at[k].at[idx_v.at[pl.ds(t * 128, 128)]], buf, sem
        )

    _issue(0, gbufs[0], gsems[0])

    obufs = (obuf0, obuf1)
    wsems = (w0, w1)

    def _wcopy16(q, ob, wsem):
        t = tg * T_PER_W + q // NCHUNK
        k = q % NCHUNK
        return pltpu.make_async_copy(
            ob.at[pl.ds(0, 16)], w_hbm.at[t, pl.ds(k * 16, 16), bb], wsem
        )

    def _wcopy13(q, ob, wsem):
        t = tg * T_PER_W + q // NCHUNK
        return pltpu.make_async_copy(
            ob.at[pl.ds(0, 13)], w_hbm.at[t, pl.ds(112, 13), bb], wsem
        )

    def body(qo):
        for kk in range(2):
            q = qo + kk
            buf = gbufs[kk]
            sem = gsems[kk]
            ob = obufs[kk]
            wsem = wsems[kk]
            t = tg * T_PER_W + q // NCHUNK
            k = q % NCHUNK
            pltpu.make_async_copy(
                tv_hbm.at[k].at[idx_v.at[pl.ds(t * 128, 128)]], buf, sem
            ).wait()
            nq = q + 1

            @pl.when(nq < NQ)
            def _():
                _issue(nq, gbufs[1 - kk], gsems[1 - kk])

            @pl.when(q >= 2)
            def _():
                k2 = (q - 2) % NCHUNK

                @pl.when(k2 < NCHUNK - 1)
                def _():
                    _wcopy16(q - 2, ob, wsem).wait()

                @pl.when(k2 == NCHUNK - 1)
                def _():
                    _wcopy13(q - 2, ob, wsem).wait()

            def tbody(cb):
                c0 = cb * 16
                col = jnp.broadcast_to(c0, (16,)).astype(jnp.int32) + lanes
                vt = jnp.broadcast_to(c0 // 8, (16,)).astype(jnp.int32) + vt0
                for b0j in range(8):
                    rows = [b0j * 16 + perms[d] for d in range(16)]
                    vals = [
                        plsc.load_gather(buf, [rows[d], col]) for d in range(16)
                    ]
                    for d in range(16):
                        plsc.store_scatter(ob, [vt, vs0, rows[d]], vals[d])

            pl.loop(0, 8)(tbody)

            @pl.when(k < NCHUNK - 1)
            def _():
                _wcopy16(q, ob, wsem).start()

            @pl.when(k == NCHUNK - 1)
            def _():
                _wcopy13(q, ob, wsem).start()

    pl.loop(0, NQ, step=2)(body)

    for q in (NQ - 2, NQ - 1):
        kk = q % 2
        if q % NCHUNK == NCHUNK - 1:
            _wcopy13(q, obufs[kk], wsems[kk]).wait()
        else:
            _wcopy16(q, obufs[kk], wsems[kk]).wait()


def kernel(idx, table):
    idx_f = (
        idx.astype(jnp.int32)
        .T.reshape(TIME, NBT, 128)
        .transpose(1, 0, 2)
        .reshape(NBT, TIME * 128)
    )
    tv = (
        jnp.pad(table, ((0, 0), (0, 24)))
        .reshape(VOCAB, NCHUNK, 128)
        .transpose(1, 0, 2)
    )
    w = _gather_t(idx_f, tv)
    return w.transpose(2, 4, 0, 1, 3).reshape(BATCH, TIME, VOCAB)


# speedup vs baseline: 1.3903x; 1.0013x over previous
import functools

import jax
import jax.numpy as jnp
from jax import lax
from jax.experimental import pallas as pl
from jax.experimental.pallas import tpu as pltpu
from jax.experimental.pallas import tpu_sc as plsc

VOCAB = 1000
BATCH, TIME = 1024, 20
NUM_CORES = 2
NUM_SUBCORES = 16
NW = NUM_CORES * NUM_SUBCORES
NBT = 8
NVT = 125
NCHUNK = 8
T_PER_W = TIME * NBT // NW
NQ = T_PER_W * NCHUNK

_MESH = plsc.VectorSubcoreMesh(core_axis_name="c", subcore_axis_name="s")


@functools.partial(
    pl.kernel,
    mesh=_MESH,
    out_type=jax.ShapeDtypeStruct((TIME, NVT, NBT, 8, 128), jnp.float32),
    scratch_types=[
        pltpu.VMEM((TIME * 128,), jnp.int32),
        pltpu.VMEM((128, 128), jnp.float32),
        pltpu.VMEM((128, 128), jnp.float32),
        pltpu.VMEM((16, 8, 128), jnp.float32),
        pltpu.VMEM((16, 8, 128), jnp.float32),
        pltpu.SemaphoreType.DMA,
        pltpu.SemaphoreType.DMA,
        pltpu.SemaphoreType.DMA,
        pltpu.SemaphoreType.DMA,
    ],
    compiler_params=pltpu.CompilerParams(
        use_tc_tiling_on_sc=False, needs_layout_passes=False
    ),
)
def _gather_t(
    idx_hbm, tv_hbm, w_hbm, idx_v, gbuf0, gbuf1, obuf0, obuf1, g0, g1, w0, w1
):
    wid = lax.axis_index("s") * NUM_CORES + lax.axis_index("c")
    bb = wid % NBT
    tg = wid // NBT
    pltpu.sync_copy(idx_hbm.at[bb], idx_v)

    lanes = lax.iota(jnp.int32, 16)
    perms = [(lanes + d) % 16 for d in range(16)]
    vt0 = lanes // 8
    vs0 = lanes % 8

    gbufs = (gbuf0, gbuf1)
    gsems = (g0, g1)

    def _issue(q, buf, sem):
        t = tg * T_PER_W + q // NCHUNK
        k = q % NCHUNK
        return pltpu.async_copy(
            tv_hbm.at[k].at[idx_v.at[pl.ds(t * 128, 128)]], buf, sem
        )

    _issue(0, gbufs[0], gsems[0])

    obufs = (obuf0, obuf1)
    wsems = (w0, w1)

    def _wcopy16(q, ob, wsem):
        t = tg * T_PER_W + q // NCHUNK
        k = q % NCHUNK
        return pltpu.make_async_copy(
            ob.at[pl.ds(0, 16)], w_hbm.at[t, pl.ds(k * 16, 16), bb], wsem
        )

    def _wcopy13(q, ob, wsem):
        t = tg * T_PER_W + q // NCHUNK
        return pltpu.make_async_copy(
            ob.at[pl.ds(0, 13)], w_hbm.at[t, pl.ds(112, 13), bb], wsem
        )

    def body(qo):
        for kk in range(2):
            q = qo + kk
            buf = gbufs[kk]
            sem = gsems[kk]
            ob = obufs[kk]
            wsem = wsems[kk]
            t = tg * T_PER_W + q // NCHUNK
            k = q % NCHUNK
            pltpu.make_async_copy(
                tv_hbm.at[k].at[idx_v.at[pl.ds(t * 128, 128)]], buf, sem
            ).wait()
            nq = q + 1

            @pl.when(nq < NQ)
            def _():
                _issue(nq, gbufs[1 - kk], gsems[1 - kk])

            @pl.when(q >= 2)
            def _():
                k2 = (q - 2) % NCHUNK

                @pl.when(k2 < NCHUNK - 1)
                def _():
                    _wcopy16(q - 2, ob, wsem).wait()

                @pl.when(k2 == NCHUNK - 1)
                def _():
                    _wcopy13(q - 2, ob, wsem).wait()

            def tbody(cb):
                c0 = cb * 16
                col = jnp.broadcast_to(c0, (16,)).astype(jnp.int32) + lanes
                vt = jnp.broadcast_to(c0 // 8, (16,)).astype(jnp.int32) + vt0
                for b0j in range(8):
                    rows = [b0j * 16 + perms[d] for d in range(16)]
                    vals = [
                        plsc.load_gather(buf, [rows[d], col]) for d in range(16)
                    ]
                    for d in range(16):
                        plsc.store_scatter(ob, [vt, vs0, rows[d]], vals[d])

            pl.loop(0, 8)(tbody)

            @pl.when(k < NCHUNK - 1)
            def _():
                _wcopy16(q, ob, wsem).start()

            @pl.when(k == NCHUNK - 1)
            def _():
                _wcopy13(q, ob, wsem).start()

    pl.loop(0, NQ, step=2)(body)

    for q in (NQ - 2, NQ - 1):
        kk = q % 2
        if q % NCHUNK == NCHUNK - 1:
            _wcopy13(q, obufs[kk], wsems[kk]).wait()
        else:
            _wcopy16(q, obufs[kk], wsems[kk]).wait()


def kernel(idx, table):
    idx_f = (
        idx.astype(jnp.int32)
        .T.reshape(TIME, NBT, 128)
        .transpose(1, 0, 2)
        .reshape(NBT, TIME * 128)
    )
    tv = (
        jnp.pad(table, ((0, 0), (0, 24)))
        .reshape(VOCAB, NCHUNK, 128)
        .transpose(1, 0, 2)
    )
    w = _gather_t(idx_f, tv)
    return w.transpose(2, 4, 0, 1, 3).reshape(BATCH, TIME, VOCAB)
